# baseline (speedup 1.0000x reference)
import dataclasses
import functools

import jax
import jax.numpy as jnp
from jax import lax
from jax.experimental import pallas as pl
from jax.experimental.pallas import tpu as pltpu
from jax.experimental.pallas import tpu_sc as plsc

D = 128
L = 16
NCU = 2
NS = 16
NW = NCU * NS
K = 128
PB = 2
P = NCU * PB

_HIGH = lax.Precision.HIGHEST


def _dot(a, b):
    return jnp.dot(a, b, preferred_element_type=jnp.float32, precision=_HIGH)


def _make_sc_bin(n_nodes, ch, bounds):
    n_pass = len(bounds)
    ept = ch * K

    mesh = plsc.VectorSubcoreMesh(core_axis_name="c", subcore_axis_name="s",
                                  num_cores=NCU)

    cp = pltpu.CompilerParams()
    if "needs_layout_passes" in pltpu.CompilerParams.__dataclass_fields__:
        cp = dataclasses.replace(cp, needs_layout_passes=False)

    @functools.partial(
        pl.kernel, mesh=mesh,
        out_type=[jax.ShapeDtypeStruct((NW, n_pass, ept), jnp.int32),
                  jax.ShapeDtypeStruct((NW, n_pass, ept), jnp.int32),
                  jax.ShapeDtypeStruct((NW, L), jnp.int32)],
        compiler_params=cp,
        scratch_types=[
            pltpu.VMEM((ept,), jnp.int32),
            pltpu.VMEM((ept,), jnp.int32),
            pltpu.VMEM((ept,), jnp.int32),
            pltpu.VMEM((ept,), jnp.int32),
            pltpu.VMEM((L,), jnp.int32),
        ])
    def bink(src_h, dst_h, sbin_o, dbin_o, cntb_o,
             sv, dv, sstg, dstg, cbuf):
        sid = lax.axis_index("c") * NS + lax.axis_index("s")

        pltpu.sync_copy(src_h.at[sid], sv)
        pltpu.sync_copy(dst_h.at[sid], dv)

        lanes = jnp.arange(L, dtype=jnp.int32)
        cnts = jnp.zeros((L,), jnp.int32)

        for p in range(n_pass):
            lo = bounds[p]
            hi = bounds[p + 1] if p + 1 < n_pass else n_nodes

            def body(g, cnt):
                d = dv[pl.ds(g * L, L)]
                s = sv[pl.ds(g * L, L)]
                m = (d >= lo) & (d < hi)
                sel = jnp.where(m, jnp.int32(1), jnp.int32(0))
                pos = plsc.cumsum(sel)
                addr = pos - 1 + cnt
                plsc.store_scatter(dstg, [addr], d, mask=m)
                plsc.store_scatter(sstg, [addr], s, mask=m)
                return cnt + jnp.sum(sel)

            cnt = lax.fori_loop(0, ept // L, body, jnp.int32(0))

            rnd = (cnt + 2 * K - 1) // (2 * K) * (2 * K)
            for g in range(2 * K // L):
                idx = cnt + g * L + lanes
                mp = idx < rnd
                plsc.store_scatter(dstg, [idx],
                                   jnp.full((L,), n_nodes, jnp.int32),
                                   mask=mp)
                plsc.store_scatter(sstg, [idx],
                                   jnp.zeros((L,), jnp.int32), mask=mp)

            cnts = jnp.where(lanes == p, rnd, cnts)

            pltpu.sync_copy(sstg, sbin_o.at[sid, p])
            pltpu.sync_copy(dstg, dbin_o.at[sid, p])

        cbuf[...] = cnts
        pltpu.sync_copy(cbuf, cntb_o.at[sid])

    return bink

def _make_sc_agg(n_nodes, na_pad, step, ch):
    rows_per_tile = na_pad // NS
    lists_per_tile = NW // NS
    ept = ch * K

    mesh = plsc.VectorSubcoreMesh(core_axis_name="c", subcore_axis_name="s",
                                  num_cores=NCU)

    @functools.partial(
        pl.kernel, mesh=mesh,
        out_type=jax.ShapeDtypeStruct((P * na_pad, D), jnp.float32),
        scratch_types=[
            pltpu.VMEM((ch, K), jnp.int32),
            pltpu.VMEM((ch, K), jnp.int32),
            pltpu.VMEM((L,), jnp.int32),
            pltpu.VMEM((1, K), jnp.int32),
            pltpu.VMEM((1, K), jnp.int32),
            pltpu.VMEM((K, D), jnp.float32),
            pltpu.VMEM((K, D), jnp.float32),
            pltpu.VMEM_SHARED((na_pad, D), jnp.float32),
            pltpu.SemaphoreType.DMA,
            pltpu.SemaphoreType.DMA,
        ])
    def agg(table_h, sbin_h, dbin_h, carr_h, zeros_h, acc_o,
            src_v, dst_v, cnt_v, dl_a, dl_b, rows_a, rows_b, acc_sh,
            sem_a, sem_b):
        cid = lax.axis_index("c")
        sid = lax.axis_index("s")
        wid = cid * NS + sid
        base = sid * rows_per_tile

        pltpu.sync_copy(carr_h.at[wid], cnt_v)
        cv = cnt_v[...]

        for pp in range(PB):
            b = cid * PB + pp
            lo = b * step
            hi = jnp.minimum(lo + step, n_nodes)

            pltpu.sync_copy(zeros_h, acc_sh.at[pl.ds(base, rows_per_tile)])

            plsc.subcore_barrier()

            def remap(j, dl_v):
                @pl.loop(0, K, step=L)
                def _(q):
                    d = dst_v[j, pl.ds(q, L)]
                    ok = (d >= lo) & (d < hi)
                    dl_v[0, pl.ds(q, L)] = jnp.where(ok, d - lo, step)

            for m in range(lists_per_tile):
                t = sid * lists_per_tile + m
                pltpu.sync_copy(sbin_h.at[t, b], src_v)
                pltpu.sync_copy(dbin_h.at[t, b], dst_v)
                nch = cv[pp * lists_per_tile + m] // K

                @pl.when(nch > 0)
                def _():
                    pltpu.async_copy(table_h.at[src_v.at[0]], rows_a, sem_a)

                @pl.loop(0, nch, step=2)
                def _(j):
                    pltpu.async_copy(table_h.at[src_v.at[j + 1]], rows_b,
                                     sem_b)
                    pltpu.make_async_copy(table_h.at[src_v.at[j]], rows_a,
                                          sem_a).wait()
                    remap(j, dl_a)
                    pltpu.sync_copy(rows_a, acc_sh.at[dl_a.at[0]], add=True)

                    @pl.when(j + 2 < nch)
                    def _():
                        pltpu.async_copy(table_h.at[src_v.at[j + 2]], rows_a,
                                         sem_a)

                    pltpu.make_async_copy(table_h.at[src_v.at[j + 1]],
                                          rows_b, sem_b).wait()
                    remap(j + 1, dl_b)
                    pltpu.sync_copy(rows_b, acc_sh.at[dl_b.at[0]], add=True)

            plsc.subcore_barrier()

            pltpu.sync_copy(
                acc_sh.at[pl.ds(base, rows_per_tile)],
                acc_o.at[pl.ds(b * na_pad + base, rows_per_tile)])

            plsc.subcore_barrier()

    return agg


def _make_sc_count(n_pad, ch):
    mesh = plsc.VectorSubcoreMesh(core_axis_name="c", subcore_axis_name="s",
                                  num_cores=NCU)

    cp = pltpu.CompilerParams()
    if "needs_layout_passes" in pltpu.CompilerParams.__dataclass_fields__:
        cp = dataclasses.replace(cp, needs_layout_passes=False)

    @functools.partial(
        pl.kernel, mesh=mesh,
        out_type=jax.ShapeDtypeStruct((NW, n_pad), jnp.float32),
        compiler_params=cp,
        scratch_types=[
            pltpu.VMEM((ch * K,), jnp.int32),
            pltpu.VMEM((n_pad,), jnp.float32),
        ])
    def count(dst_h, cnt_o, dst_v, hist_v):
        wid = lax.axis_index("c") * NS + lax.axis_index("s")

        pltpu.sync_copy(dst_h.at[wid], dst_v)

        @pl.loop(0, n_pad, step=L)
        def _(i):
            hist_v[pl.ds(i, L)] = jnp.zeros((L,), jnp.float32)

        @pl.loop(0, ch * K, step=L)
        def _(i):
            d = dst_v[pl.ds(i, L)]
            plsc.addupdate_scatter(hist_v, [d], jnp.ones((L,), jnp.float32))

        pltpu.sync_copy(hist_v, cnt_o.at[wid])

    return count


_BN = 400


def _wspec():
    return pl.BlockSpec((D, D), lambda i: (0, 0))


def _bspec():
    return pl.BlockSpec((1, D), lambda i: (0, 0))


def _nspec(width=D):
    return pl.BlockSpec((_BN, width), lambda i: (i, 0))


def _tc_embed(x, W_embed, b_embed, W1):
    n = x.shape[0]

    def body(x_ref, we_ref, be_ref, w1_ref, h0_ref, hw1_ref):
        h0 = _dot(x_ref[...], we_ref[...]) + be_ref[...]
        h0_ref[...] = h0
        hw1_ref[...] = _dot(h0, w1_ref[...])

    return pl.pallas_call(
        body,
        grid=(n // _BN,),
        in_specs=[_nspec(), _wspec(), _bspec(), _wspec()],
        out_specs=[_nspec(), _nspec()],
        out_shape=[jax.ShapeDtypeStruct((n, D), jnp.float32)] * 2,
    )(x, W_embed, b_embed, W1)


def _tc_combine(acc, cnt, h_prev, root, b, W_next, b_next, final):
    n = h_prev.shape[0]

    def body(a_ref, c_ref, hp_ref, root_ref, b_ref, wn_ref, bn_ref,
             *out_refs):
        cnt = jnp.sum(c_ref[...], axis=1, keepdims=True)
        inv = 1.0 / jnp.maximum(cnt, 1.0)
        agg = a_ref[...] * inv
        h = jnp.maximum(agg + _dot(hp_ref[...], root_ref[...]) + b_ref[...],
                        0.0)
        if final:
            out_refs[0][...] = _dot(h, wn_ref[...]) + bn_ref[...]
        else:
            out_refs[0][...] = h
            out_refs[1][...] = _dot(h, wn_ref[...])

    n_out = 1 if final else 2
    res = pl.pallas_call(
        body,
        grid=(n // _BN,),
        in_specs=[_nspec(), _nspec(NW), _nspec(),
                  _wspec(), _bspec(), _wspec(), _bspec()],
        out_specs=[_nspec()] * n_out,
        out_shape=[jax.ShapeDtypeStruct((n, D), jnp.float32)] * n_out,
    )(acc, cnt, h_prev, root, b, W_next, b_next)
    return res[0] if final else res


def kernel(x, edge_index, W_embed, b_embed, W1, root1, b1, W2, root2, b2,
           Wf, bf):
    n, _ = x.shape
    e = edge_index.shape[1]

    ept = -(-e // (NW * 2 * K)) * 2 * K
    ch = ept // K
    e_pad = ept * NW
    n_pad = -(-n // (NS * K)) * (NS * K)
    step = -(-n // P)
    na_pad = -(-(step + 1) // (NS * 8)) * (NS * 8)

    src = edge_index[0]
    dst = edge_index[1]
    pad = e_pad - e
    if pad:
        src = jnp.concatenate([src, jnp.zeros((pad,), src.dtype)])
        dst = jnp.concatenate([dst, jnp.full((pad,), n, dst.dtype)])
    src_f = src.reshape(NW, ch * K)
    dst_f = dst.reshape(NW, ch * K)

    b_embed = b_embed.reshape(1, D)
    b1r = b1.reshape(1, D)
    b2r = b2.reshape(1, D)
    bfr = bf.reshape(1, D)

    bink = _make_sc_bin(n, ch, bounds=[p * step for p in range(P)])
    agg = _make_sc_agg(n, na_pad, step, ch)
    count = _make_sc_count(n_pad, ch)

    def assemble(acc):
        return jnp.concatenate(
            [acc[b * na_pad: b * na_pad + min(step, n - b * step)]
             for b in range(P)])

    h0, hw1 = _tc_embed(x, W_embed, b_embed, W1)

    zeros = jnp.zeros((na_pad // NS, D), jnp.float32)

    sbin, dbin, cntb = bink(src_f, dst_f)
    sbin = sbin.reshape(NW, P, ch, K)
    dbin = dbin.reshape(NW, P, ch, K)

    lists = NW // NS
    tidx = [[(w % NS) * lists + (i % lists) if i < PB * lists else 0
             for i in range(L)] for w in range(NW)]
    bidx = [[(w // NS) * PB + (i // lists) if i < PB * lists else 0
             for i in range(L)] for w in range(NW)]
    carr = cntb[jnp.array(tidx), jnp.array(bidx)]

    cnt_t = jnp.transpose(count(dst_f))[:n]
    acc1 = assemble(agg(hw1, sbin, dbin, carr, zeros))

    h1, hw2 = _tc_combine(acc1, cnt_t, h0, root1, b1r, W2, b1r,
                          final=False)

    acc2 = assemble(agg(hw2, sbin, dbin, carr, zeros))

    out = _tc_combine(acc2, cnt_t, h1, root2, b2r, Wf, bfr,
                      final=True)
    return out


# speedup vs baseline: 1.3528x; 1.3528x over previous
import dataclasses
import functools

import jax
import jax.numpy as jnp
from jax import lax
from jax.experimental import pallas as pl
from jax.experimental.pallas import tpu as pltpu
from jax.experimental.pallas import tpu_sc as plsc

D = 128
L = 16
NCU = 1
NS = 16
NW = NCU * NS
K = 128
PB = 4
P = NCU * PB

_HIGH = lax.Precision.HIGHEST


def _dot(a, b):
    return jnp.dot(a, b, preferred_element_type=jnp.float32, precision=_HIGH)


def _make_sc_bin(n_nodes, ch, bounds):
    n_pass = len(bounds)
    ept = ch * K

    mesh = plsc.VectorSubcoreMesh(core_axis_name="c", subcore_axis_name="s",
                                  num_cores=NCU)

    cp = pltpu.CompilerParams()
    if "needs_layout_passes" in pltpu.CompilerParams.__dataclass_fields__:
        cp = dataclasses.replace(cp, needs_layout_passes=False)

    @functools.partial(
        pl.kernel, mesh=mesh,
        out_type=[jax.ShapeDtypeStruct((NW, n_pass, ept), jnp.int32),
                  jax.ShapeDtypeStruct((NW, n_pass, ept), jnp.int32),
                  jax.ShapeDtypeStruct((NW, L), jnp.int32)],
        compiler_params=cp,
        scratch_types=[
            pltpu.VMEM((ept,), jnp.int32),
            pltpu.VMEM((ept,), jnp.int32),
            pltpu.VMEM((ept,), jnp.int32),
            pltpu.VMEM((ept,), jnp.int32),
            pltpu.VMEM((L,), jnp.int32),
        ])
    def bink(src_h, dst_h, sbin_o, dbin_o, cntb_o,
             sv, dv, sstg, dstg, cbuf):
        sid = lax.axis_index("c") * NS + lax.axis_index("s")

        pltpu.sync_copy(src_h.at[sid], sv)
        pltpu.sync_copy(dst_h.at[sid], dv)

        lanes = jnp.arange(L, dtype=jnp.int32)
        cnts = jnp.zeros((L,), jnp.int32)

        for p in range(n_pass):
            lo = bounds[p]
            hi = bounds[p + 1] if p + 1 < n_pass else n_nodes

            def body(g, cnt):
                d = dv[pl.ds(g * L, L)]
                s = sv[pl.ds(g * L, L)]
                m = (d >= lo) & (d < hi)
                sel = jnp.where(m, jnp.int32(1), jnp.int32(0))
                pos = plsc.cumsum(sel)
                addr = pos - 1 + cnt
                plsc.store_scatter(dstg, [addr], d, mask=m)
                plsc.store_scatter(sstg, [addr], s, mask=m)
                return cnt + jnp.sum(sel)

            cnt = lax.fori_loop(0, ept // L, body, jnp.int32(0))

            rnd = (cnt + 2 * K - 1) // (2 * K) * (2 * K)
            for g in range(2 * K // L):
                idx = cnt + g * L + lanes
                mp = idx < rnd
                plsc.store_scatter(dstg, [idx],
                                   jnp.full((L,), n_nodes, jnp.int32),
                                   mask=mp)
                plsc.store_scatter(sstg, [idx],
                                   jnp.zeros((L,), jnp.int32), mask=mp)

            cnts = jnp.where(lanes == p, rnd, cnts)

            pltpu.sync_copy(sstg, sbin_o.at[sid, p])
            pltpu.sync_copy(dstg, dbin_o.at[sid, p])

        cbuf[...] = cnts
        pltpu.sync_copy(cbuf, cntb_o.at[sid])

    return bink

def _make_sc_agg(n_nodes, na_pad, step, ch):
    rows_per_tile = na_pad // NS
    lists_per_tile = NW // NS
    ept = ch * K

    mesh = plsc.VectorSubcoreMesh(core_axis_name="c", subcore_axis_name="s",
                                  num_cores=NCU)

    @functools.partial(
        pl.kernel, mesh=mesh,
        out_type=jax.ShapeDtypeStruct((P * na_pad, D), jnp.float32),
        scratch_types=[
            pltpu.VMEM((ch, K), jnp.int32),
            pltpu.VMEM((ch, K), jnp.int32),
            pltpu.VMEM((L,), jnp.int32),
            pltpu.VMEM((1, K), jnp.int32),
            pltpu.VMEM((1, K), jnp.int32),
            pltpu.VMEM((K, D), jnp.float32),
            pltpu.VMEM((K, D), jnp.float32),
            pltpu.VMEM_SHARED((na_pad, D), jnp.float32),
            pltpu.SemaphoreType.DMA,
            pltpu.SemaphoreType.DMA,
        ])
    def agg(table_h, sbin_h, dbin_h, carr_h, zeros_h, acc_o,
            src_v, dst_v, cnt_v, dl_a, dl_b, rows_a, rows_b, acc_sh,
            sem_a, sem_b):
        cid = lax.axis_index("c")
        sid = lax.axis_index("s")
        wid = cid * NS + sid
        base = sid * rows_per_tile

        pltpu.sync_copy(carr_h.at[wid], cnt_v)
        cv = cnt_v[...]

        for pp in range(PB):
            b = cid * PB + pp
            lo = b * step
            hi = jnp.minimum(lo + step, n_nodes)

            pltpu.sync_copy(zeros_h, acc_sh.at[pl.ds(base, rows_per_tile)])

            plsc.subcore_barrier()

            def remap(j, dl_v):
                @pl.loop(0, K, step=L)
                def _(q):
                    d = dst_v[j, pl.ds(q, L)]
                    ok = (d >= lo) & (d < hi)
                    dl_v[0, pl.ds(q, L)] = jnp.where(ok, d - lo, step)

            for m in range(lists_per_tile):
                t = sid * lists_per_tile + m
                pltpu.sync_copy(sbin_h.at[t, b], src_v)
                pltpu.sync_copy(dbin_h.at[t, b], dst_v)
                nch = cv[pp * lists_per_tile + m] // K

                @pl.when(nch > 0)
                def _():
                    pltpu.async_copy(table_h.at[src_v.at[0]], rows_a, sem_a)

                @pl.loop(0, nch, step=2)
                def _(j):
                    pltpu.async_copy(table_h.at[src_v.at[j + 1]], rows_b,
                                     sem_b)
                    pltpu.make_async_copy(table_h.at[src_v.at[j]], rows_a,
                                          sem_a).wait()
                    remap(j, dl_a)
                    pltpu.sync_copy(rows_a, acc_sh.at[dl_a.at[0]], add=True)

                    @pl.when(j + 2 < nch)
                    def _():
                        pltpu.async_copy(table_h.at[src_v.at[j + 2]], rows_a,
                                         sem_a)

                    pltpu.make_async_copy(table_h.at[src_v.at[j + 1]],
                                          rows_b, sem_b).wait()
                    remap(j + 1, dl_b)
                    pltpu.sync_copy(rows_b, acc_sh.at[dl_b.at[0]], add=True)

            plsc.subcore_barrier()

            pltpu.sync_copy(
                acc_sh.at[pl.ds(base, rows_per_tile)],
                acc_o.at[pl.ds(b * na_pad + base, rows_per_tile)])

            plsc.subcore_barrier()

    return agg


def _make_sc_count(n_pad, ch):
    mesh = plsc.VectorSubcoreMesh(core_axis_name="c", subcore_axis_name="s",
                                  num_cores=NCU)

    cp = pltpu.CompilerParams()
    if "needs_layout_passes" in pltpu.CompilerParams.__dataclass_fields__:
        cp = dataclasses.replace(cp, needs_layout_passes=False)

    @functools.partial(
        pl.kernel, mesh=mesh,
        out_type=jax.ShapeDtypeStruct((NW, n_pad), jnp.float32),
        compiler_params=cp,
        scratch_types=[
            pltpu.VMEM((ch * K,), jnp.int32),
            pltpu.VMEM((n_pad,), jnp.float32),
        ])
    def count(dst_h, cnt_o, dst_v, hist_v):
        wid = lax.axis_index("c") * NS + lax.axis_index("s")

        pltpu.sync_copy(dst_h.at[wid], dst_v)

        @pl.loop(0, n_pad, step=L)
        def _(i):
            hist_v[pl.ds(i, L)] = jnp.zeros((L,), jnp.float32)

        @pl.loop(0, ch * K, step=L)
        def _(i):
            d = dst_v[pl.ds(i, L)]
            plsc.addupdate_scatter(hist_v, [d], jnp.ones((L,), jnp.float32))

        pltpu.sync_copy(hist_v, cnt_o.at[wid])

    return count


_BN = 400


def _wspec():
    return pl.BlockSpec((D, D), lambda i: (0, 0))


def _bspec():
    return pl.BlockSpec((1, D), lambda i: (0, 0))


def _nspec(width=D):
    return pl.BlockSpec((_BN, width), lambda i: (i, 0))


def _tc_embed(x, W_embed, b_embed, W1):
    n = x.shape[0]

    def body(x_ref, we_ref, be_ref, w1_ref, h0_ref, hw1_ref):
        h0 = _dot(x_ref[...], we_ref[...]) + be_ref[...]
        h0_ref[...] = h0
        hw1_ref[...] = _dot(h0, w1_ref[...])

    return pl.pallas_call(
        body,
        grid=(n // _BN,),
        in_specs=[_nspec(), _wspec(), _bspec(), _wspec()],
        out_specs=[_nspec(), _nspec()],
        out_shape=[jax.ShapeDtypeStruct((n, D), jnp.float32)] * 2,
    )(x, W_embed, b_embed, W1)


def _tc_combine(acc, cnt, h_prev, root, b, W_next, b_next, final):
    n = h_prev.shape[0]

    def body(a_ref, c_ref, hp_ref, root_ref, b_ref, wn_ref, bn_ref,
             *out_refs):
        cnt = jnp.sum(c_ref[...], axis=1, keepdims=True)
        inv = 1.0 / jnp.maximum(cnt, 1.0)
        agg = a_ref[...] * inv
        h = jnp.maximum(agg + _dot(hp_ref[...], root_ref[...]) + b_ref[...],
                        0.0)
        if final:
            out_refs[0][...] = _dot(h, wn_ref[...]) + bn_ref[...]
        else:
            out_refs[0][...] = h
            out_refs[1][...] = _dot(h, wn_ref[...])

    n_out = 1 if final else 2
    res = pl.pallas_call(
        body,
        grid=(n // _BN,),
        in_specs=[_nspec(), _nspec(NW), _nspec(),
                  _wspec(), _bspec(), _wspec(), _bspec()],
        out_specs=[_nspec()] * n_out,
        out_shape=[jax.ShapeDtypeStruct((n, D), jnp.float32)] * n_out,
    )(acc, cnt, h_prev, root, b, W_next, b_next)
    return res[0] if final else res


def kernel(x, edge_index, W_embed, b_embed, W1, root1, b1, W2, root2, b2,
           Wf, bf):
    n, _ = x.shape
    e = edge_index.shape[1]

    ept = -(-e // (NW * 2 * K)) * 2 * K
    ch = ept // K
    e_pad = ept * NW
    n_pad = -(-n // (NS * K)) * (NS * K)
    step = -(-n // P)
    na_pad = -(-(step + 1) // (NS * 8)) * (NS * 8)

    src = edge_index[0]
    dst = edge_index[1]
    pad = e_pad - e
    if pad:
        src = jnp.concatenate([src, jnp.zeros((pad,), src.dtype)])
        dst = jnp.concatenate([dst, jnp.full((pad,), n, dst.dtype)])
    src_f = src.reshape(NW, ch * K)
    dst_f = dst.reshape(NW, ch * K)

    b_embed = b_embed.reshape(1, D)
    b1r = b1.reshape(1, D)
    b2r = b2.reshape(1, D)
    bfr = bf.reshape(1, D)

    bink = _make_sc_bin(n, ch, bounds=[p * step for p in range(P)])
    agg = _make_sc_agg(n, na_pad, step, ch)
    count = _make_sc_count(n_pad, ch)

    def assemble(acc):
        return jnp.concatenate(
            [acc[b * na_pad: b * na_pad + min(step, n - b * step)]
             for b in range(P)])

    h0, hw1 = _tc_embed(x, W_embed, b_embed, W1)

    zeros = jnp.zeros((na_pad // NS, D), jnp.float32)

    sbin, dbin, cntb = bink(src_f, dst_f)
    sbin = sbin.reshape(NW, P, ch, K)
    dbin = dbin.reshape(NW, P, ch, K)

    lists = NW // NS
    tidx = [[(w % NS) * lists + (i % lists) if i < PB * lists else 0
             for i in range(L)] for w in range(NW)]
    bidx = [[(w // NS) * PB + (i // lists) if i < PB * lists else 0
             for i in range(L)] for w in range(NW)]
    carr = cntb[jnp.array(tidx), jnp.array(bidx)]

    cnt_t = jnp.transpose(count(dst_f))[:n]
    acc1 = assemble(agg(hw1, sbin, dbin, carr, zeros))

    h1, hw2 = _tc_combine(acc1, cnt_t, h0, root1, b1r, W2, b1r,
                          final=False)

    acc2 = assemble(agg(hw2, sbin, dbin, carr, zeros))

    out = _tc_combine(acc2, cnt_t, h1, root2, b2r, Wf, bfr,
                      final=True)
    return out


# speedup vs baseline: 1.8197x; 1.3452x over previous
import dataclasses
import functools

import jax
import jax.numpy as jnp
from jax import lax
from jax.experimental import pallas as pl
from jax.experimental.pallas import tpu as pltpu
from jax.experimental.pallas import tpu_sc as plsc

D = 128
L = 16
NCU = 1
NS = 16
NW = NCU * NS
K = 128

_HIGH = lax.Precision.HIGHEST


def _dot(a, b):
    return jnp.dot(a, b, preferred_element_type=jnp.float32, precision=_HIGH)


def _make_sc_bin(n_nodes, ch, bounds):
    n_pass = len(bounds)
    ept = ch * K
    dummy = max(b_hi - b_lo
                for b_lo, b_hi in zip(bounds, bounds[1:] + [n_nodes]))

    mesh = plsc.VectorSubcoreMesh(core_axis_name="c", subcore_axis_name="s",
                                  num_cores=NCU)

    cp = pltpu.CompilerParams()
    if "needs_layout_passes" in pltpu.CompilerParams.__dataclass_fields__:
        cp = dataclasses.replace(cp, needs_layout_passes=False)

    @functools.partial(
        pl.kernel, mesh=mesh,
        out_type=[jax.ShapeDtypeStruct((NW, n_pass, ch, K), jnp.int32),
                  jax.ShapeDtypeStruct((NW, n_pass, ch, K), jnp.int32),
                  jax.ShapeDtypeStruct((NW, L), jnp.int32)],
        compiler_params=cp,
        scratch_types=[
            pltpu.VMEM((ept,), jnp.int32),
            pltpu.VMEM((ept,), jnp.int32),
            pltpu.VMEM((ch, K), jnp.int32),
            pltpu.VMEM((ch, K), jnp.int32),
            pltpu.VMEM((L,), jnp.int32),
        ])
    def bink(src_h, dst_h, sbin_o, dbin_o, cntb_o,
             sv, dv, sstg, dstg, cbuf):
        sid = lax.axis_index("s")

        pltpu.sync_copy(src_h.at[sid], sv)
        pltpu.sync_copy(dst_h.at[sid], dv)

        lanes = jnp.arange(L, dtype=jnp.int32)
        cnts = jnp.zeros((L,), jnp.int32)

        for p in range(n_pass):
            lo = bounds[p]
            hi = bounds[p + 1] if p + 1 < n_pass else n_nodes

            def body(g, cnt):
                d = dv[pl.ds(g * L, L)]
                s = sv[pl.ds(g * L, L)]
                m = (d >= lo) & (d < hi)
                sel = jnp.where(m, jnp.int32(1), jnp.int32(0))
                pos = plsc.cumsum(sel)
                addr = pos - 1 + cnt
                plsc.store_scatter(dstg, [addr >> 7, addr & 127], d - lo,
                                   mask=m)
                plsc.store_scatter(sstg, [addr >> 7, addr & 127], s, mask=m)
                return cnt + jnp.sum(sel)

            cnt = lax.fori_loop(0, ept // L, body, jnp.int32(0))

            rnd = (cnt + 2 * K - 1) // (2 * K) * (2 * K)
            for g in range(2 * K // L):
                idx = cnt + g * L + lanes
                mp = idx < rnd
                plsc.store_scatter(dstg, [idx >> 7, idx & 127],
                                   jnp.full((L,), dummy, jnp.int32),
                                   mask=mp)
                plsc.store_scatter(sstg, [idx >> 7, idx & 127],
                                   jnp.zeros((L,), jnp.int32), mask=mp)

            cnts = jnp.where(lanes == p, rnd, cnts)

            pltpu.sync_copy(sstg, sbin_o.at[sid, p])
            pltpu.sync_copy(dstg, dbin_o.at[sid, p])

        cbuf[...] = cnts
        pltpu.sync_copy(cbuf, cntb_o.at[sid])

    return bink

def _make_sc_agg(n_nodes, na_pad, ch, bounds):
    rows_per_tile = na_pad // NS
    nz = rows_per_tile // K
    n_pass = len(bounds)
    dummy = max(b_hi - b_lo for b_lo, b_hi in zip(bounds, bounds[1:] + [n_nodes]))

    mesh = plsc.VectorSubcoreMesh(core_axis_name="c", subcore_axis_name="s",
                                  num_cores=NCU)

    @functools.partial(
        pl.kernel, mesh=mesh,
        out_type=jax.ShapeDtypeStruct((n_pass * na_pad, D), jnp.float32),
        scratch_types=[
            pltpu.VMEM((ch, K), jnp.int32),
            pltpu.VMEM((ch, K), jnp.int32),
            pltpu.VMEM((L,), jnp.int32),
            pltpu.VMEM((K, D), jnp.float32),
            pltpu.VMEM((K, D), jnp.float32),
            pltpu.VMEM_SHARED((na_pad, D), jnp.float32),
            pltpu.SemaphoreType.DMA,
            pltpu.SemaphoreType.DMA,
        ])
    def agg(table_h, sbin_h, dbin_h, cntb_h, acc_o,
            src_v, dst_v, cnt_v, rows_a, rows_b, acc_sh,
            sem_a, sem_b):
        sid = lax.axis_index("s")
        base = sid * rows_per_tile

        pltpu.sync_copy(cntb_h.at[sid], cnt_v)

        for p in range(n_pass):
            pltpu.sync_copy(sbin_h.at[sid, p], src_v)
            pltpu.sync_copy(dbin_h.at[sid, p], dst_v)

            nch = cnt_v[...][p] // K

            @pl.loop(0, K)
            def _(i):
                @pl.loop(0, D, step=L)
                def _(j):
                    rows_a[i, pl.ds(j, L)] = jnp.zeros((L,), jnp.float32)

            @pl.loop(0, nz)
            def _(kk):
                pltpu.sync_copy(rows_a, acc_sh.at[pl.ds(base + kk * K, K)])

            plsc.subcore_barrier()

            @pl.when(nch > 0)
            def _():
                pltpu.async_copy(table_h.at[src_v.at[0]], rows_a, sem_a)

            @pl.loop(0, nch, step=2)
            def _(j):
                pltpu.async_copy(table_h.at[src_v.at[j + 1]], rows_b, sem_b)
                pltpu.make_async_copy(table_h.at[src_v.at[j]], rows_a,
                                      sem_a).wait()
                pltpu.sync_copy(rows_a, acc_sh.at[dst_v.at[j]], add=True)

                @pl.when(j + 2 < nch)
                def _():
                    pltpu.async_copy(table_h.at[src_v.at[j + 2]], rows_a,
                                     sem_a)

                pltpu.make_async_copy(table_h.at[src_v.at[j + 1]], rows_b,
                                      sem_b).wait()
                pltpu.sync_copy(rows_b, acc_sh.at[dst_v.at[j + 1]], add=True)

            plsc.subcore_barrier()

            pltpu.sync_copy(acc_sh.at[pl.ds(base, rows_per_tile)],
                            acc_o.at[pl.ds(p * na_pad + base, rows_per_tile)])

            plsc.subcore_barrier()

    return agg


def _make_sc_count(n_pad, ch):
    mesh = plsc.VectorSubcoreMesh(core_axis_name="c", subcore_axis_name="s",
                                  num_cores=NCU)

    cp = pltpu.CompilerParams()
    if "needs_layout_passes" in pltpu.CompilerParams.__dataclass_fields__:
        cp = dataclasses.replace(cp, needs_layout_passes=False)

    @functools.partial(
        pl.kernel, mesh=mesh,
        out_type=jax.ShapeDtypeStruct((NS, n_pad), jnp.float32),
        compiler_params=cp,
        scratch_types=[
            pltpu.VMEM((ch * K,), jnp.int32),
            pltpu.VMEM((n_pad,), jnp.float32),
        ])
    def count(dst_h, cnt_o, dst_v, hist_v):
        sid = lax.axis_index("s")

        pltpu.sync_copy(dst_h.at[sid], dst_v)

        @pl.loop(0, n_pad, step=L)
        def _(i):
            hist_v[pl.ds(i, L)] = jnp.zeros((L,), jnp.float32)

        @pl.loop(0, ch * K, step=L)
        def _(i):
            d = dst_v[pl.ds(i, L)]
            plsc.addupdate_scatter(hist_v, [d], jnp.ones((L,), jnp.float32))

        pltpu.sync_copy(hist_v, cnt_o.at[sid])

    return count


_BN = 400


def _wspec():
    return pl.BlockSpec((D, D), lambda i: (0, 0))


def _bspec():
    return pl.BlockSpec((1, D), lambda i: (0, 0))


def _nspec(width=D):
    return pl.BlockSpec((_BN, width), lambda i: (i, 0))


def _tc_embed(x, W_embed, b_embed, W1):
    n = x.shape[0]

    def body(x_ref, we_ref, be_ref, w1_ref, h0_ref, hw1_ref):
        h0 = _dot(x_ref[...], we_ref[...]) + be_ref[...]
        h0_ref[...] = h0
        hw1_ref[...] = _dot(h0, w1_ref[...])

    return pl.pallas_call(
        body,
        grid=(n // _BN,),
        in_specs=[_nspec(), _wspec(), _bspec(), _wspec()],
        out_specs=[_nspec(), _nspec()],
        out_shape=[jax.ShapeDtypeStruct((n, D), jnp.float32)] * 2,
    )(x, W_embed, b_embed, W1)


def _tc_combine(acc, cnt, h_prev, root, b, W_next, b_next, final):
    n = h_prev.shape[0]

    def body(a_ref, c_ref, hp_ref, root_ref, b_ref, wn_ref, bn_ref,
             *out_refs):
        cnt = jnp.sum(c_ref[...], axis=1, keepdims=True)
        inv = 1.0 / jnp.maximum(cnt, 1.0)
        agg = a_ref[...] * inv
        h = jnp.maximum(agg + _dot(hp_ref[...], root_ref[...]) + b_ref[...],
                        0.0)
        if final:
            out_refs[0][...] = _dot(h, wn_ref[...]) + bn_ref[...]
        else:
            out_refs[0][...] = h
            out_refs[1][...] = _dot(h, wn_ref[...])

    n_out = 1 if final else 2
    res = pl.pallas_call(
        body,
        grid=(n // _BN,),
        in_specs=[_nspec(), _nspec(NS), _nspec(),
                  _wspec(), _bspec(), _wspec(), _bspec()],
        out_specs=[_nspec()] * n_out,
        out_shape=[jax.ShapeDtypeStruct((n, D), jnp.float32)] * n_out,
    )(acc, cnt, h_prev, root, b, W_next, b_next)
    return res[0] if final else res


def kernel(x, edge_index, W_embed, b_embed, W1, root1, b1, W2, root2, b2,
           Wf, bf):
    n, _ = x.shape
    e = edge_index.shape[1]

    ept = -(-e // (NW * 2 * K)) * 2 * K
    ch = ept // K
    e_pad = ept * NW
    n_pad = -(-n // (NS * K)) * (NS * K)
    na = (n // 2) // K * K
    na_pad = -(-(max(na, n - na) + 1) // (NS * K)) * (NS * K)

    src = edge_index[0]
    dst = edge_index[1]
    pad = e_pad - e
    if pad:
        src = jnp.concatenate([src, jnp.zeros((pad,), src.dtype)])
        dst = jnp.concatenate([dst, jnp.full((pad,), n, dst.dtype)])
    src_f = src.reshape(NW, ch * K)
    dst_f = dst.reshape(NW, ch * K)

    b_embed = b_embed.reshape(1, D)
    b1r = b1.reshape(1, D)
    b2r = b2.reshape(1, D)
    bfr = bf.reshape(1, D)

    bink = _make_sc_bin(n, ch, bounds=[0, na])
    agg = _make_sc_agg(n, na_pad, ch, bounds=[0, na])
    count = _make_sc_count(n_pad, ch)

    def assemble(acc):
        return jnp.concatenate([acc[:na], acc[na_pad:na_pad + (n - na)]])

    h0, hw1 = _tc_embed(x, W_embed, b_embed, W1)

    sbin, dbin, cntb = bink(src_f, dst_f)
    cnt_t = jnp.transpose(count(dst_f))[:n]
    acc1 = assemble(agg(hw1, sbin, dbin, cntb))

    h1, hw2 = _tc_combine(acc1, cnt_t, h0, root1, b1r, W2, b1r,
                          final=False)

    acc2 = assemble(agg(hw2, sbin, dbin, cntb))

    out = _tc_combine(acc2, cnt_t, h1, root2, b2r, Wf, bfr,
                      final=True)
    return out


# speedup vs baseline: 1.8510x; 1.0172x over previous
import dataclasses
import functools

import jax
import jax.numpy as jnp
from jax import lax
from jax.experimental import pallas as pl
from jax.experimental.pallas import tpu as pltpu
from jax.experimental.pallas import tpu_sc as plsc

D = 128
L = 16
NCU = 1
NS = 16
NW = NCU * NS
K = 128

_HIGH = lax.Precision.HIGHEST


def _dot(a, b):
    return jnp.dot(a, b, preferred_element_type=jnp.float32, precision=_HIGH)


def _make_sc_bin(n_nodes, ch, bounds):
    n_pass = len(bounds)
    ept = ch * K
    dummy = max(b_hi - b_lo
                for b_lo, b_hi in zip(bounds, bounds[1:] + [n_nodes]))

    mesh = plsc.VectorSubcoreMesh(core_axis_name="c", subcore_axis_name="s",
                                  num_cores=NCU)

    cp = pltpu.CompilerParams()
    if "needs_layout_passes" in pltpu.CompilerParams.__dataclass_fields__:
        cp = dataclasses.replace(cp, needs_layout_passes=False)

    @functools.partial(
        pl.kernel, mesh=mesh,
        out_type=[jax.ShapeDtypeStruct((NW, n_pass, ch, K), jnp.int32),
                  jax.ShapeDtypeStruct((NW, n_pass, ch, K), jnp.int32),
                  jax.ShapeDtypeStruct((NW, L), jnp.int32)],
        compiler_params=cp,
        scratch_types=[
            pltpu.VMEM((ept,), jnp.int32),
            pltpu.VMEM((ept,), jnp.int32),
            pltpu.VMEM((ch, K), jnp.int32),
            pltpu.VMEM((ch, K), jnp.int32),
            pltpu.VMEM((L,), jnp.int32),
        ])
    def bink(src_h, dst_h, sbin_o, dbin_o, cntb_o,
             sv, dv, sstg, dstg, cbuf):
        sid = lax.axis_index("s")

        pltpu.sync_copy(src_h.at[sid], sv)
        pltpu.sync_copy(dst_h.at[sid], dv)

        lanes = jnp.arange(L, dtype=jnp.int32)
        cnts = jnp.zeros((L,), jnp.int32)

        for p in range(n_pass):
            lo = bounds[p]
            hi = bounds[p + 1] if p + 1 < n_pass else n_nodes

            def body(g, cnt):
                d = dv[pl.ds(g * L, L)]
                s = sv[pl.ds(g * L, L)]
                m = (d >= lo) & (d < hi)
                sel = jnp.where(m, jnp.int32(1), jnp.int32(0))
                pos = plsc.cumsum(sel)
                addr = pos - 1 + cnt
                plsc.store_scatter(dstg, [addr >> 7, addr & 127], d - lo,
                                   mask=m)
                plsc.store_scatter(sstg, [addr >> 7, addr & 127], s, mask=m)
                return cnt + jnp.sum(sel)

            cnt = lax.fori_loop(0, ept // L, body, jnp.int32(0))

            rnd = (cnt + 2 * K - 1) // (2 * K) * (2 * K)
            for g in range(2 * K // L):
                idx = cnt + g * L + lanes
                mp = idx < rnd
                plsc.store_scatter(dstg, [idx >> 7, idx & 127],
                                   jnp.full((L,), dummy, jnp.int32),
                                   mask=mp)
                plsc.store_scatter(sstg, [idx >> 7, idx & 127],
                                   jnp.zeros((L,), jnp.int32), mask=mp)

            cnts = jnp.where(lanes == p, rnd, cnts)

            pltpu.sync_copy(sstg, sbin_o.at[sid, p])
            pltpu.sync_copy(dstg, dbin_o.at[sid, p])

        cbuf[...] = cnts
        pltpu.sync_copy(cbuf, cntb_o.at[sid])

    return bink

def _make_sc_agg(n_nodes, na_pad, ch, bounds):
    rows_per_tile = na_pad // NS
    nz = rows_per_tile // K
    n_pass = len(bounds)
    dummy = max(b_hi - b_lo for b_lo, b_hi in zip(bounds, bounds[1:] + [n_nodes]))

    mesh = plsc.VectorSubcoreMesh(core_axis_name="c", subcore_axis_name="s",
                                  num_cores=NCU)

    @functools.partial(
        pl.kernel, mesh=mesh,
        out_type=jax.ShapeDtypeStruct((n_pass * na_pad, D), jnp.float32),
        scratch_types=[
            pltpu.VMEM((ch, K), jnp.int32),
            pltpu.VMEM((ch, K), jnp.int32),
            pltpu.VMEM((L,), jnp.int32),
            pltpu.VMEM((K, D), jnp.float32),
            pltpu.VMEM((K, D), jnp.float32),
            pltpu.VMEM_SHARED((na_pad, D), jnp.float32),
            pltpu.SemaphoreType.DMA,
            pltpu.SemaphoreType.DMA,
        ])
    def agg(table_h, sbin_h, dbin_h, cntb_h, acc_o,
            src_v, dst_v, cnt_v, rows_a, rows_b, acc_sh,
            sem_a, sem_b):
        sid = lax.axis_index("s")
        base = sid * rows_per_tile

        pltpu.sync_copy(cntb_h.at[sid], cnt_v)

        for p in range(n_pass):
            pltpu.sync_copy(sbin_h.at[sid, p], src_v)
            pltpu.sync_copy(dbin_h.at[sid, p], dst_v)

            nch = cnt_v[...][p] // K

            @pl.loop(0, K)
            def _(i):
                @pl.loop(0, D, step=L)
                def _(j):
                    rows_a[i, pl.ds(j, L)] = jnp.zeros((L,), jnp.float32)

            @pl.loop(0, nz)
            def _(kk):
                pltpu.sync_copy(rows_a, acc_sh.at[pl.ds(base + kk * K, K)])

            plsc.subcore_barrier()

            @pl.when(nch > 0)
            def _():
                pltpu.async_copy(table_h.at[src_v.at[0]], rows_a, sem_a)

            @pl.loop(0, nch, step=2)
            def _(j):
                pltpu.async_copy(table_h.at[src_v.at[j + 1]], rows_b, sem_b)
                pltpu.make_async_copy(table_h.at[src_v.at[j]], rows_a,
                                      sem_a).wait()
                pltpu.sync_copy(rows_a, acc_sh.at[dst_v.at[j]], add=True)

                @pl.when(j + 2 < nch)
                def _():
                    pltpu.async_copy(table_h.at[src_v.at[j + 2]], rows_a,
                                     sem_a)

                pltpu.make_async_copy(table_h.at[src_v.at[j + 1]], rows_b,
                                      sem_b).wait()
                pltpu.sync_copy(rows_b, acc_sh.at[dst_v.at[j + 1]], add=True)

            plsc.subcore_barrier()

            pltpu.sync_copy(acc_sh.at[pl.ds(base, rows_per_tile)],
                            acc_o.at[pl.ds(p * na_pad + base, rows_per_tile)])

            plsc.subcore_barrier()

    return agg


def _make_sc_count(n_pad, ch):
    mesh = plsc.VectorSubcoreMesh(core_axis_name="c", subcore_axis_name="s",
                                  num_cores=NCU)

    cp = pltpu.CompilerParams()
    if "needs_layout_passes" in pltpu.CompilerParams.__dataclass_fields__:
        cp = dataclasses.replace(cp, needs_layout_passes=False)

    @functools.partial(
        pl.kernel, mesh=mesh,
        out_type=jax.ShapeDtypeStruct((NS, n_pad), jnp.float32),
        compiler_params=cp,
        scratch_types=[
            pltpu.VMEM((ch * K,), jnp.int32),
            pltpu.VMEM((n_pad,), jnp.float32),
        ])
    def count(dst_h, cnt_o, dst_v, hist_v):
        sid = lax.axis_index("s")

        pltpu.sync_copy(dst_h.at[sid], dst_v)

        @pl.loop(0, n_pad, step=L)
        def _(i):
            hist_v[pl.ds(i, L)] = jnp.zeros((L,), jnp.float32)

        @pl.loop(0, ch * K, step=L)
        def _(i):
            d = dst_v[pl.ds(i, L)]
            plsc.addupdate_scatter(hist_v, [d], jnp.ones((L,), jnp.float32))

        pltpu.sync_copy(hist_v, cnt_o.at[sid])

    return count


_BN = 2000


def _wspec():
    return pl.BlockSpec((D, D), lambda i: (0, 0))


def _bspec():
    return pl.BlockSpec((1, D), lambda i: (0, 0))


def _nspec(width=D):
    return pl.BlockSpec((_BN, width), lambda i: (i, 0))


def _tc_embed(x, W_embed, b_embed, W1):
    n = x.shape[0]

    def body(x_ref, we_ref, be_ref, w1_ref, h0_ref, hw1_ref):
        h0 = _dot(x_ref[...], we_ref[...]) + be_ref[...]
        h0_ref[...] = h0
        hw1_ref[...] = _dot(h0, w1_ref[...])

    return pl.pallas_call(
        body,
        grid=(n // _BN,),
        in_specs=[_nspec(), _wspec(), _bspec(), _wspec()],
        out_specs=[_nspec(), _nspec()],
        out_shape=[jax.ShapeDtypeStruct((n, D), jnp.float32)] * 2,
    )(x, W_embed, b_embed, W1)


def _tc_combine(acc, cnt, h_prev, root, b, W_next, b_next, final):
    n = h_prev.shape[0]

    def body(a_ref, c_ref, hp_ref, root_ref, b_ref, wn_ref, bn_ref,
             *out_refs):
        cnt = jnp.sum(c_ref[...], axis=1, keepdims=True)
        inv = 1.0 / jnp.maximum(cnt, 1.0)
        agg = a_ref[...] * inv
        h = jnp.maximum(agg + _dot(hp_ref[...], root_ref[...]) + b_ref[...],
                        0.0)
        if final:
            out_refs[0][...] = _dot(h, wn_ref[...]) + bn_ref[...]
        else:
            out_refs[0][...] = h
            out_refs[1][...] = _dot(h, wn_ref[...])

    n_out = 1 if final else 2
    res = pl.pallas_call(
        body,
        grid=(n // _BN,),
        in_specs=[_nspec(), _nspec(NS), _nspec(),
                  _wspec(), _bspec(), _wspec(), _bspec()],
        out_specs=[_nspec()] * n_out,
        out_shape=[jax.ShapeDtypeStruct((n, D), jnp.float32)] * n_out,
    )(acc, cnt, h_prev, root, b, W_next, b_next)
    return res[0] if final else res


def kernel(x, edge_index, W_embed, b_embed, W1, root1, b1, W2, root2, b2,
           Wf, bf):
    n, _ = x.shape
    e = edge_index.shape[1]

    ept = -(-e // (NW * 2 * K)) * 2 * K
    ch = ept // K
    e_pad = ept * NW
    n_pad = -(-n // (NS * K)) * (NS * K)
    na = (n // 2) // K * K
    na_pad = -(-(max(na, n - na) + 1) // (NS * K)) * (NS * K)

    src = edge_index[0]
    dst = edge_index[1]
    pad = e_pad - e
    if pad:
        src = jnp.concatenate([src, jnp.zeros((pad,), src.dtype)])
        dst = jnp.concatenate([dst, jnp.full((pad,), n, dst.dtype)])
    src_f = src.reshape(NW, ch * K)
    dst_f = dst.reshape(NW, ch * K)

    b_embed = b_embed.reshape(1, D)
    b1r = b1.reshape(1, D)
    b2r = b2.reshape(1, D)
    bfr = bf.reshape(1, D)

    bink = _make_sc_bin(n, ch, bounds=[0, na])
    agg = _make_sc_agg(n, na_pad, ch, bounds=[0, na])
    count = _make_sc_count(n_pad, ch)

    def assemble(acc):
        return jnp.concatenate([acc[:na], acc[na_pad:na_pad + (n - na)]])

    h0, hw1 = _tc_embed(x, W_embed, b_embed, W1)

    sbin, dbin, cntb = bink(src_f, dst_f)
    cnt_t = jnp.transpose(count(dst_f))[:n]
    acc1 = assemble(agg(hw1, sbin, dbin, cntb))

    h1, hw2 = _tc_combine(acc1, cnt_t, h0, root1, b1r, W2, b1r,
                          final=False)

    acc2 = assemble(agg(hw2, sbin, dbin, cntb))

    out = _tc_combine(acc2, cnt_t, h1, root2, b2r, Wf, bfr,
                      final=True)
    return out
